# baseline (speedup 1.0000x reference)
import functools

import jax
import jax.numpy as jnp
from jax import lax
from jax.experimental import pallas as pl
from jax.experimental.pallas import tpu as pltpu
from jax.experimental.pallas import tpu_sc as plsc

_CHUNK = 448
_R = 128
_TC_CHUNKS = 12

_SORT8_PAIRS = [(0, 1), (2, 3), (4, 5), (6, 7),
                (0, 2), (1, 3), (4, 6), (5, 7),
                (1, 2), (5, 6), (0, 4), (3, 7),
                (1, 5), (2, 6),
                (1, 4), (3, 6),
                (2, 4), (3, 5),
                (3, 4)]

_BITONIC_STAGES = [[(0, 4), (1, 5), (2, 6), (3, 7)],
                   [(0, 2), (1, 3), (4, 6), (5, 7)],
                   [(0, 1), (2, 3), (4, 5), (6, 7)]]


def _cex(v, i, j):
    hi = jnp.maximum(v[i], v[j])
    lo = jnp.minimum(v[i], v[j])
    v[i] = hi
    v[j] = lo


def _sort8(vs):
    vs = list(vs)
    for i, j in _SORT8_PAIRS:
        _cex(vs, i, j)
    return vs


def _merge8(a, b):
    v = [jnp.maximum(a[i], b[7 - i]) for i in range(8)]
    for stage in _BITONIC_STAGES:
        for i, j in stage:
            _cex(v, i, j)
    return v


def _tree_reduce(vs):
    r = vs[0].shape[0]
    while r > 1:
        h = r // 2
        m = _merge8([v[:h] for v in vs], [v[h:2 * h] for v in vs])
        if r % 2:
            vs = [jnp.concatenate([m[k], vs[k][2 * h:]], axis=0)
                  for k in range(8)]
        else:
            vs = m
        r = h + (r % 2)
    return vs


def _make_tc_leaf(nch):
    def leaf(x_hbm, o_ref, buf, cand, sem0, sem1):
        sems = (sem0, sem1)
        b = o_ref.shape[0]
        npair = (b * nch) // 2

        def copy(ci, slot):
            bi = ci // nch
            ri = ci % nch
            return pltpu.make_async_copy(
                x_hbm.at[bi, pl.ds(ri * _CHUNK, _CHUNK), :],
                buf.at[slot], sems[slot])

        def process(ci, slot):
            x = buf[slot]
            g = _CHUNK // 8
            vs = _tree_reduce(_sort8([x[g * j:g * (j + 1)]
                                      for j in range(8)]))
            cand[ci % nch] = jnp.concatenate(vs, axis=0)

            @pl.when(ci % nch == nch - 1)
            def _():
                fin = _tree_reduce([cand[:, k, :] for k in range(8)])
                o_ref[ci // nch] = jnp.concatenate(fin, axis=0)

        copy(0, 0).start()

        def pair(p, _):
            ci = p * 2
            copy(ci + 1, 1).start()
            copy(ci, 0).wait()
            process(ci, 0)

            @pl.when(p < npair - 1)
            def _():
                copy(ci + 2, 0).start()

            copy(ci + 1, 1).wait()
            process(ci + 1, 1)
            return 0

        lax.fori_loop(0, npair, pair, 0)

    return leaf


def _combine_kernel(x_ref, y_ref, o_ref):
    vs = _merge8([x_ref[:, k, :] for k in range(8)],
                 [y_ref[:, k, :] for k in range(8)])
    for k in range(8):
        o_ref[:, k, :] = vs[k]


def _sc_topk(inputs, l0):
    b, l, c = inputs.shape
    info = plsc.get_sparse_core_info()
    nw = info.num_cores * info.num_subcores
    lanes = info.num_lanes
    nslab = nw // b
    cw = c // nslab
    ng = cw // lanes
    nch = (l - l0) // _R
    mesh = plsc.VectorSubcoreMesh(core_axis_name="c", subcore_axis_name="s")

    @functools.partial(
        pl.kernel, mesh=mesh,
        out_type=jax.ShapeDtypeStruct((b, 8, c), jnp.float32),
        scratch_types=[
            pltpu.VMEM((2, _R, cw), jnp.float32),
            pltpu.VMEM((8, cw), jnp.float32),
            pltpu.SemaphoreType.DMA,
            pltpu.SemaphoreType.DMA,
        ],
    )
    def run(x_hbm, out_hbm, buf, st, sem0, sem1):
        wid = lax.axis_index("s") * info.num_cores + lax.axis_index("c")
        bi = wid // nslab
        c0 = (wid % nslab) * cw

        def chunk_slice(ci):
            return x_hbm.at[bi, pl.ds(l0 + ci * _R, _R), pl.ds(c0, cw)]

        def process(slot):
            for g in range(ng):
                lo = g * lanes
                s = tuple(st[k, lo:lo + lanes] for k in range(8))

                def step(j, s):
                    base = j * 8
                    v = [buf[slot, base + i, lo:lo + lanes] for i in range(8)]
                    return tuple(_merge8(list(s), _sort8(v)))

                s = lax.fori_loop(0, _R // 8, step, s)
                for k in range(8):
                    st[k, lo:lo + lanes] = s[k]

        neg = jnp.full((lanes,), -jnp.inf, jnp.float32)
        for g in range(ng):
            for k in range(8):
                st[k, g * lanes:(g + 1) * lanes] = neg
        pltpu.async_copy(chunk_slice(0), buf.at[0], sem0)

        def pair(j, _):
            ci = j * 2
            pltpu.async_copy(chunk_slice(ci + 1), buf.at[1], sem1)
            pltpu.make_async_copy(chunk_slice(ci), buf.at[0], sem0).wait()
            process(0)

            @pl.when(j < nch // 2 - 1)
            def _():
                pltpu.async_copy(chunk_slice(ci + 2), buf.at[0], sem0)

            pltpu.make_async_copy(chunk_slice(ci + 1), buf.at[1], sem1).wait()
            process(1)
            return 0

        lax.fori_loop(0, nch // 2, pair, 0)
        pltpu.sync_copy(st, out_hbm.at[bi, :, pl.ds(c0, cw)])

    return run(inputs)


def kernel(inputs):
    b, l, c = inputs.shape
    sc_cand = _sc_topk(inputs, _TC_CHUNKS * _CHUNK)
    tc_cand = pl.pallas_call(
        _make_tc_leaf(_TC_CHUNKS),
        grid=(1,),
        in_specs=[pl.BlockSpec(memory_space=pltpu.MemorySpace.HBM)],
        out_specs=pl.BlockSpec((b, 8, c), lambda i: (0, 0, 0)),
        out_shape=jax.ShapeDtypeStruct((b, 8, c), inputs.dtype),
        scratch_shapes=[
            pltpu.VMEM((2, _CHUNK, c), inputs.dtype),
            pltpu.VMEM((_TC_CHUNKS, 8, c), inputs.dtype),
            pltpu.SemaphoreType.DMA,
            pltpu.SemaphoreType.DMA,
        ],
    )(inputs)
    return pl.pallas_call(
        _combine_kernel,
        in_specs=[pl.BlockSpec((b, 8, c), lambda: (0, 0, 0)),
                  pl.BlockSpec((b, 8, c), lambda: (0, 0, 0))],
        out_specs=pl.BlockSpec((b, 8, c), lambda: (0, 0, 0)),
        out_shape=jax.ShapeDtypeStruct((b, 8, c), inputs.dtype),
    )(tc_cand, sc_cand)


# speedup vs baseline: 1.0603x; 1.0603x over previous
import functools

import jax
import jax.numpy as jnp
from jax import lax
from jax.experimental import pallas as pl
from jax.experimental.pallas import tpu as pltpu
from jax.experimental.pallas import tpu_sc as plsc

_CHUNK = 512
_R = 128
_TC_CHUNKS = 11

_SORT8_PAIRS = [(0, 1), (2, 3), (4, 5), (6, 7),
                (0, 2), (1, 3), (4, 6), (5, 7),
                (1, 2), (5, 6), (0, 4), (3, 7),
                (1, 5), (2, 6),
                (1, 4), (3, 6),
                (2, 4), (3, 5),
                (3, 4)]

_BITONIC_STAGES = [[(0, 4), (1, 5), (2, 6), (3, 7)],
                   [(0, 2), (1, 3), (4, 6), (5, 7)],
                   [(0, 1), (2, 3), (4, 5), (6, 7)]]


def _cex(v, i, j):
    hi = jnp.maximum(v[i], v[j])
    lo = jnp.minimum(v[i], v[j])
    v[i] = hi
    v[j] = lo


def _sort8(vs):
    vs = list(vs)
    for i, j in _SORT8_PAIRS:
        _cex(vs, i, j)
    return vs


def _merge8(a, b):
    v = [jnp.maximum(a[i], b[7 - i]) for i in range(8)]
    for stage in _BITONIC_STAGES:
        for i, j in stage:
            _cex(v, i, j)
    return v


def _tree_reduce(vs):
    r = vs[0].shape[0]
    while r > 1:
        h = r // 2
        m = _merge8([v[:h] for v in vs], [v[h:2 * h] for v in vs])
        if r % 2:
            vs = [jnp.concatenate([m[k], vs[k][2 * h:]], axis=0)
                  for k in range(8)]
        else:
            vs = m
        r = h + (r % 2)
    return vs


def _make_tc_leaf(nch):
    def leaf(x_hbm, o_ref, buf, cand, sem0, sem1):
        sems = (sem0, sem1)
        b = o_ref.shape[0]
        npair = (b * nch) // 2

        def copy(ci, slot):
            bi = ci // nch
            ri = ci % nch
            return pltpu.make_async_copy(
                x_hbm.at[bi, pl.ds(ri * _CHUNK, _CHUNK), :],
                buf.at[slot], sems[slot])

        def process(ci, slot):
            x = buf[slot]
            g = _CHUNK // 8
            vs = _tree_reduce(_sort8([x[g * j:g * (j + 1)]
                                      for j in range(8)]))
            cand[ci % nch] = jnp.concatenate(vs, axis=0)

            @pl.when(ci % nch == nch - 1)
            def _():
                fin = _tree_reduce([cand[:, k, :] for k in range(8)])
                o_ref[ci // nch] = jnp.concatenate(fin, axis=0)

        copy(0, 0).start()

        def pair(p, _):
            ci = p * 2
            copy(ci + 1, 1).start()
            copy(ci, 0).wait()
            process(ci, 0)

            @pl.when(p < npair - 1)
            def _():
                copy(ci + 2, 0).start()

            copy(ci + 1, 1).wait()
            process(ci + 1, 1)
            return 0

        lax.fori_loop(0, npair, pair, 0)

    return leaf


def _combine_kernel(x_ref, y_ref, o_ref):
    vs = _merge8([x_ref[:, k, :] for k in range(8)],
                 [y_ref[:, k, :] for k in range(8)])
    for k in range(8):
        o_ref[:, k, :] = vs[k]


def _sc_topk(inputs, l0):
    b, l, c = inputs.shape
    info = plsc.get_sparse_core_info()
    nw = info.num_cores * info.num_subcores
    lanes = info.num_lanes
    nslab = nw // b
    cw = c // nslab
    ng = cw // lanes
    nch = (l - l0) // _R
    mesh = plsc.VectorSubcoreMesh(core_axis_name="c", subcore_axis_name="s")

    @functools.partial(
        pl.kernel, mesh=mesh,
        out_type=jax.ShapeDtypeStruct((b, 8, c), jnp.float32),
        scratch_types=[
            pltpu.VMEM((2, _R, cw), jnp.float32),
            pltpu.VMEM((8, cw), jnp.float32),
            pltpu.SemaphoreType.DMA,
            pltpu.SemaphoreType.DMA,
        ],
    )
    def run(x_hbm, out_hbm, buf, st, sem0, sem1):
        wid = lax.axis_index("s") * info.num_cores + lax.axis_index("c")
        bi = wid // nslab
        c0 = (wid % nslab) * cw

        def chunk_slice(ci):
            return x_hbm.at[bi, pl.ds(l0 + ci * _R, _R), pl.ds(c0, cw)]

        def process(slot):
            for g in range(ng):
                lo = g * lanes
                s = tuple(st[k, lo:lo + lanes] for k in range(8))

                def step(j, s):
                    base = j * 8
                    v = [buf[slot, base + i, lo:lo + lanes] for i in range(8)]
                    return tuple(_merge8(list(s), _sort8(v)))

                s = lax.fori_loop(0, _R // 8, step, s)
                for k in range(8):
                    st[k, lo:lo + lanes] = s[k]

        neg = jnp.full((lanes,), -jnp.inf, jnp.float32)
        for g in range(ng):
            for k in range(8):
                st[k, g * lanes:(g + 1) * lanes] = neg
        pltpu.async_copy(chunk_slice(0), buf.at[0], sem0)

        def pair(j, _):
            ci = j * 2
            pltpu.async_copy(chunk_slice(ci + 1), buf.at[1], sem1)
            pltpu.make_async_copy(chunk_slice(ci), buf.at[0], sem0).wait()
            process(0)

            @pl.when(j < nch // 2 - 1)
            def _():
                pltpu.async_copy(chunk_slice(ci + 2), buf.at[0], sem0)

            pltpu.make_async_copy(chunk_slice(ci + 1), buf.at[1], sem1).wait()
            process(1)
            return 0

        lax.fori_loop(0, nch // 2, pair, 0)
        pltpu.sync_copy(st, out_hbm.at[bi, :, pl.ds(c0, cw)])

    return run(inputs)


def kernel(inputs):
    b, l, c = inputs.shape
    sc_cand = _sc_topk(inputs, _TC_CHUNKS * _CHUNK)
    tc_cand = pl.pallas_call(
        _make_tc_leaf(_TC_CHUNKS),
        grid=(1,),
        in_specs=[pl.BlockSpec(memory_space=pltpu.MemorySpace.HBM)],
        out_specs=pl.BlockSpec((b, 8, c), lambda i: (0, 0, 0)),
        out_shape=jax.ShapeDtypeStruct((b, 8, c), inputs.dtype),
        scratch_shapes=[
            pltpu.VMEM((2, _CHUNK, c), inputs.dtype),
            pltpu.VMEM((_TC_CHUNKS, 8, c), inputs.dtype),
            pltpu.SemaphoreType.DMA,
            pltpu.SemaphoreType.DMA,
        ],
    )(inputs)
    return pl.pallas_call(
        _combine_kernel,
        in_specs=[pl.BlockSpec((b, 8, c), lambda: (0, 0, 0)),
                  pl.BlockSpec((b, 8, c), lambda: (0, 0, 0))],
        out_specs=pl.BlockSpec((b, 8, c), lambda: (0, 0, 0)),
        out_shape=jax.ShapeDtypeStruct((b, 8, c), inputs.dtype),
    )(tc_cand, sc_cand)
